# baseline (speedup 1.0000x reference)
import functools

import jax
import jax.numpy as jnp
from jax import lax
from jax.experimental import pallas as pl
from jax.experimental.pallas import tpu as pltpu
from jax.experimental.pallas import tpu_sc as plsc

NN = 2048
NE = 65536
NB = 64
DIN = 8064
DH = 5000
SL = 2500
EPT = NE // 32
ASZ = NB * NN


def _sc_body(src_hbm, dst_hbm, w_hbm, batch_hbm, out_hbm,
             batch_v, srcb, dstb, key2d, w2d, zbuf, a_sh):
    c = lax.axis_index("c")
    s = lax.axis_index("s")
    wid = c * 16 + s

    def _zb(i, carry):
        zbuf[pl.ds(i * 16, 16)] = jnp.zeros((16,), jnp.float32)
        return carry
    lax.fori_loop(0, 64, _zb, 0)
    for j in range(8):
        pltpu.sync_copy(zbuf, a_sh.at[pl.ds(s * 8192 + j * 1024, 1024)])

    base = wid * EPT
    pltpu.sync_copy(src_hbm.at[pl.ds(base, EPT)], srcb)
    pltpu.sync_copy(dst_hbm.at[pl.ds(base, EPT)], dstb)
    pltpu.sync_copy(batch_hbm, batch_v)
    for j in range(16):
        pltpu.sync_copy(w_hbm.at[pl.ds(base + j * 128, 128)], w2d.at[j])

    for j in range(16):
        for i in range(8):
            off = j * 128 + i * 16
            d16 = dstb[pl.ds(off, 16)]
            s16 = srcb[pl.ds(off, 16)]
            bd = plsc.load_gather(batch_v, [d16])
            key2d[j, pl.ds(i * 16, 16)] = bd * NN + s16

    plsc.subcore_barrier()
    for j in range(16):
        pltpu.sync_copy(w2d.at[j], a_sh.at[key2d.at[j]], add=True)
    plsc.subcore_barrier()

    pltpu.sync_copy(a_sh.at[pl.ds(s * 8192, 8192)],
                    out_hbm.at[c, pl.ds(s * 8192, 8192)])


@functools.cache
def _build_edge_matrix():
    return functools.partial(
        pl.kernel,
        out_type=jax.ShapeDtypeStruct((2, ASZ), jnp.float32),
        mesh=plsc.VectorSubcoreMesh(core_axis_name="c", subcore_axis_name="s"),
        compiler_params=pltpu.CompilerParams(needs_layout_passes=False),
        scratch_types=[
            pltpu.VMEM((NN,), jnp.int32),
            pltpu.VMEM((EPT,), jnp.int32),
            pltpu.VMEM((EPT,), jnp.int32),
            pltpu.VMEM((16, 128), jnp.int32),
            pltpu.VMEM((16, 128), jnp.float32),
            pltpu.VMEM((1024,), jnp.float32),
            pltpu.VMEM_SHARED((ASZ,), jnp.float32),
        ],
    )(_sc_body)


XB = 1152


def _pool_mm_body(a_ref, b_ref, x_ref, p_ref, q_ref, cnt_ref):
    amat = a_ref[0] + a_ref[1]
    brow = b_ref[0:1, :]
    rows = lax.broadcasted_iota(jnp.int32, (NB, NN), 0)
    oh = (brow == rows).astype(jnp.float32)
    xb = x_ref[...]
    p_ref[...] = jnp.dot(amat, xb, preferred_element_type=jnp.float32)
    q_ref[...] = jnp.dot(oh, xb, preferred_element_type=jnp.float32)

    @pl.when(pl.program_id(0) == 0)
    def _():
        cnt_ref[...] = jnp.broadcast_to(
            jnp.sum(oh, axis=1, keepdims=True), (NB, 128))


def _pool_mm(a2, batch2d, x):
    return pl.pallas_call(
        _pool_mm_body,
        grid=(DIN // XB,),
        in_specs=[
            pl.BlockSpec((2, NB, NN), lambda j: (0, 0, 0)),
            pl.BlockSpec((8, NN), lambda j: (0, 0)),
            pl.BlockSpec((NN, XB), lambda j: (0, j)),
        ],
        out_specs=[
            pl.BlockSpec((NB, XB), lambda j: (0, j)),
            pl.BlockSpec((NB, XB), lambda j: (0, j)),
            pl.BlockSpec((NB, 128), lambda j: (0, 0)),
        ],
        out_shape=[
            jax.ShapeDtypeStruct((NB, DIN), jnp.float32),
            jax.ShapeDtypeStruct((NB, DIN), jnp.float32),
            jax.ShapeDtypeStruct((NB, 128), jnp.float32),
        ],
    )(a2, batch2d, x)


KB = 384


HB = 256


def _head_mm_body(p_ref, q_ref, wr_ref, wo_ref, b1_ref, cnt_ref, o_ref):
    acc = jnp.dot(p_ref[...], wr_ref[...], preferred_element_type=jnp.float32)
    acc += jnp.dot(q_ref[...], wo_ref[...], preferred_element_type=jnp.float32)
    o_ref[...] = acc + cnt_ref[:, 0:1] * b1_ref[...]


def _head_mm(p, q, wrel, wroot, b1row, cnt):
    nblk = pl.cdiv(DH, HB)
    return pl.pallas_call(
        _head_mm_body,
        grid=(nblk,),
        compiler_params=pltpu.CompilerParams(
            dimension_semantics=("parallel",)),
        in_specs=[
            pl.BlockSpec((NB, DIN), lambda j: (0, 0)),
            pl.BlockSpec((NB, DIN), lambda j: (0, 0)),
            pl.BlockSpec((DIN, HB), lambda j: (0, j)),
            pl.BlockSpec((DIN, HB), lambda j: (0, j)),
            pl.BlockSpec((1, HB), lambda j: (0, j)),
            pl.BlockSpec((NB, 128), lambda j: (0, 0)),
        ],
        out_specs=pl.BlockSpec((NB, HB), lambda j: (0, j)),
        out_shape=jax.ShapeDtypeStruct((NB, DH), jnp.float32),
    )(p, q, wrel, wroot, b1row, cnt)


def _lstm_step(xt, h, c, wi, wh, b, nin):
    g = b
    for i in range(nin):
        g = g + wi[i] * xt[i:i + 1, :]
    for k in range(3):
        g = g + wh[k] * h[k:k + 1, :]
    sg = jax.nn.sigmoid(g)
    gi = sg[0:3]
    gf = sg[3:6]
    gg = jnp.tanh(g[6:9])
    go = sg[9:12]
    c2 = gf * c + gi * gg
    h2 = go * jnp.tanh(c2)
    return h2, c2


def _lstm0_body(x_ref, wi_ref, wh_ref, b_ref, o_ref):
    wi = [wi_ref[i] for i in range(2)]
    wh = [wh_ref[k] for k in range(3)]
    b = b_ref[...]

    def step(t, carry):
        h, c = carry
        h2, c2 = _lstm_step(x_ref[t], h, c, wi, wh, b, 2)
        o_ref[t] = h2
        return (h2, c2)

    z = jnp.zeros((3, 128), jnp.float32)
    lax.fori_loop(0, SL, step, (z, z))


def _lstm1_body(x_ref, wi_ref, wh_ref, b_ref, mw_ref, mb_ref, o_ref):
    wi = [wi_ref[i] for i in range(6)]
    wh = [wh_ref[k] for k in range(3)]
    b = b_ref[...]

    def step(t, carry):
        h, c, acc = carry
        h2, c2 = _lstm_step(x_ref[t], h, c, wi, wh, b, 6)
        return (h2, c2, acc + h2 * mw_ref[t])

    z = jnp.zeros((3, 128), jnp.float32)
    _, _, acc = lax.fori_loop(0, SL, step, (z, z, z))
    tot = acc[0:1] + acc[1:2] + acc[2:3]
    o_ref[...] = tot[:, 0:64] + tot[:, 64:128] + mb_ref[...]


def _lstm0(xcat, wi, wh, b):
    return pl.pallas_call(
        _lstm0_body,
        out_shape=jax.ShapeDtypeStruct((SL, 3, 128), jnp.float32),
    )(xcat, wi, wh, b)


def _lstm1(xcat, wi, wh, b, mw, mb):
    return pl.pallas_call(
        _lstm1_body,
        out_shape=jax.ShapeDtypeStruct((1, 64), jnp.float32),
    )(xcat, wi, wh, b, mw, mb)


def _dir_pair(pf, pr, key, rows):
    f = jnp.broadcast_to(pf[key].T[:, :, None], (pf[key].shape[1], rows, 64))
    r = jnp.broadcast_to(pr[key].T[:, :, None], (pr[key].shape[1], rows, 64))
    return jnp.concatenate([f, r], axis=2)


def _bias_pair(pf, pr):
    bf = jnp.broadcast_to((pf["bih"] + pf["bhh"])[:, None], (12, 64))
    br = jnp.broadcast_to((pr["bih"] + pr["bhh"])[:, None], (12, 64))
    return jnp.concatenate([bf, br], axis=1)


def kernel(x, edge_index, edge_attr, batch, params):
    src = edge_index[0].astype(jnp.int32)
    dst = edge_index[1].astype(jnp.int32)
    batch = batch.astype(jnp.int32)

    a2 = _build_edge_matrix()(src, dst, edge_attr, batch)
    return _tail(a2, x, batch, params)


def _tail(a2, x, batch, params):
    a2 = a2.reshape(2, NB, NN)

    batch2d = jnp.broadcast_to(batch[None, :], (8, NN))
    p, q, cnt = _pool_mm(a2, batch2d, x)
    pooled = _head_mm(p, q, params["W1_rel"], params["W1_root"],
                      params["b1_rel"][None, :], cnt)

    seq = pooled.reshape(NB, SL, 2).transpose(1, 2, 0)
    xcat0 = jnp.concatenate([seq, seq[::-1]], axis=2)

    lp = params["lstm"]
    wi0 = _dir_pair(lp["l0"]["fwd"], lp["l0"]["rev"], "Wih", 12)
    wh0 = _dir_pair(lp["l0"]["fwd"], lp["l0"]["rev"], "Whh", 12)
    b0 = _bias_pair(lp["l0"]["fwd"], lp["l0"]["rev"])
    out0 = _lstm0(xcat0, wi0, wh0, b0)

    f0 = out0[:, :, 0:64]
    r0 = out0[::-1, :, 64:128]
    in1 = jnp.concatenate([f0, r0], axis=1)
    xcat1 = jnp.concatenate([in1, in1[::-1]], axis=2)

    wi1 = _dir_pair(lp["l1"]["fwd"], lp["l1"]["rev"], "Wih", 12)
    wh1 = _dir_pair(lp["l1"]["fwd"], lp["l1"]["rev"], "Whh", 12)
    b1 = _bias_pair(lp["l1"]["fwd"], lp["l1"]["rev"])

    m6 = params["mlp_W"].reshape(SL, 6)
    mwf = jnp.broadcast_to(m6[:, 0:3][:, :, None], (SL, 3, 64))
    mwr = jnp.broadcast_to(m6[::-1, 3:6][:, :, None], (SL, 3, 64))
    mw = jnp.concatenate([mwf, mwr], axis=2)
    mb = jnp.broadcast_to(params["mlp_b"][None, :], (1, 64))

    res = _lstm1(xcat1, wi1, wh1, b1, mw, mb)
    return res.reshape(NB, 1)


# speedup vs baseline: 1.0770x; 1.0770x over previous
import functools

import jax
import jax.numpy as jnp
from jax import lax
from jax.experimental import pallas as pl
from jax.experimental.pallas import tpu as pltpu
from jax.experimental.pallas import tpu_sc as plsc

NN = 2048
NE = 65536
NB = 64
DIN = 8064
DH = 5000
SL = 2500
EPT = NE // 32
ASZ = NB * NN


def _sc_body(src_hbm, dst_hbm, w_hbm, batch_hbm, out_hbm,
             batch_v, srcb, dstb, key2d, w2d, zbuf, a_sh):
    c = lax.axis_index("c")
    s = lax.axis_index("s")
    wid = c * 16 + s

    def _zb(i, carry):
        zbuf[pl.ds(i * 16, 16)] = jnp.zeros((16,), jnp.float32)
        return carry
    lax.fori_loop(0, 64, _zb, 0)
    for j in range(8):
        pltpu.sync_copy(zbuf, a_sh.at[pl.ds(s * 8192 + j * 1024, 1024)])

    base = wid * EPT
    pltpu.sync_copy(src_hbm.at[pl.ds(base, EPT)], srcb)
    pltpu.sync_copy(dst_hbm.at[pl.ds(base, EPT)], dstb)
    pltpu.sync_copy(batch_hbm, batch_v)
    for j in range(16):
        pltpu.sync_copy(w_hbm.at[pl.ds(base + j * 128, 128)], w2d.at[j])

    for j in range(16):
        for i in range(8):
            off = j * 128 + i * 16
            d16 = dstb[pl.ds(off, 16)]
            s16 = srcb[pl.ds(off, 16)]
            bd = plsc.load_gather(batch_v, [d16])
            key2d[j, pl.ds(i * 16, 16)] = bd * NN + s16

    plsc.subcore_barrier()
    for j in range(16):
        pltpu.sync_copy(w2d.at[j], a_sh.at[key2d.at[j]], add=True)
    plsc.subcore_barrier()

    pltpu.sync_copy(a_sh.at[pl.ds(s * 8192, 8192)],
                    out_hbm.at[c, pl.ds(s * 8192, 8192)])


@functools.cache
def _build_edge_matrix():
    return functools.partial(
        pl.kernel,
        out_type=jax.ShapeDtypeStruct((2, ASZ), jnp.float32),
        mesh=plsc.VectorSubcoreMesh(core_axis_name="c", subcore_axis_name="s"),
        compiler_params=pltpu.CompilerParams(needs_layout_passes=False),
        scratch_types=[
            pltpu.VMEM((NN,), jnp.int32),
            pltpu.VMEM((EPT,), jnp.int32),
            pltpu.VMEM((EPT,), jnp.int32),
            pltpu.VMEM((16, 128), jnp.int32),
            pltpu.VMEM((16, 128), jnp.float32),
            pltpu.VMEM((1024,), jnp.float32),
            pltpu.VMEM_SHARED((ASZ,), jnp.float32),
        ],
    )(_sc_body)


XB = 1152


def _pool_mm_body(a_ref, b_ref, x_ref, p_ref, q_ref, cnt_ref):
    amat = a_ref[0] + a_ref[1]
    brow = b_ref[0:1, :]
    rows = lax.broadcasted_iota(jnp.int32, (NB, NN), 0)
    oh = (brow == rows).astype(jnp.float32)
    xb = x_ref[...]
    p_ref[...] = jnp.dot(amat, xb, preferred_element_type=jnp.float32)
    q_ref[...] = jnp.dot(oh, xb, preferred_element_type=jnp.float32)

    @pl.when(pl.program_id(0) == 0)
    def _():
        cnt_ref[...] = jnp.broadcast_to(
            jnp.sum(oh, axis=1, keepdims=True), (NB, 128))


def _pool_mm(a2, batch2d, x):
    return pl.pallas_call(
        _pool_mm_body,
        grid=(DIN // XB,),
        in_specs=[
            pl.BlockSpec((2, NB, NN), lambda j: (0, 0, 0)),
            pl.BlockSpec((8, NN), lambda j: (0, 0)),
            pl.BlockSpec((NN, XB), lambda j: (0, j)),
        ],
        out_specs=[
            pl.BlockSpec((NB, XB), lambda j: (0, j)),
            pl.BlockSpec((NB, XB), lambda j: (0, j)),
            pl.BlockSpec((NB, 128), lambda j: (0, 0)),
        ],
        out_shape=[
            jax.ShapeDtypeStruct((NB, DIN), jnp.float32),
            jax.ShapeDtypeStruct((NB, DIN), jnp.float32),
            jax.ShapeDtypeStruct((NB, 128), jnp.float32),
        ],
    )(a2, batch2d, x)


KB = 384


HB = 256


def _head_mm_body(p_ref, q_ref, wr_ref, wo_ref, b1_ref, cnt_ref, o_ref):
    acc = jnp.dot(p_ref[...], wr_ref[...], preferred_element_type=jnp.float32)
    acc += jnp.dot(q_ref[...], wo_ref[...], preferred_element_type=jnp.float32)
    o_ref[...] = (acc + cnt_ref[:, 0:1] * b1_ref[...]).T


def _head_mm(p, q, wrel, wroot, b1row, cnt):
    nblk = pl.cdiv(DH, HB)
    return pl.pallas_call(
        _head_mm_body,
        grid=(nblk,),
        in_specs=[
            pl.BlockSpec((NB, DIN), lambda j: (0, 0)),
            pl.BlockSpec((NB, DIN), lambda j: (0, 0)),
            pl.BlockSpec((DIN, HB), lambda j: (0, j)),
            pl.BlockSpec((DIN, HB), lambda j: (0, j)),
            pl.BlockSpec((1, HB), lambda j: (0, j)),
            pl.BlockSpec((NB, 128), lambda j: (0, 0)),
        ],
        out_specs=pl.BlockSpec((HB, NB), lambda j: (j, 0)),
        out_shape=jax.ShapeDtypeStruct((DH, NB), jnp.float32),
    )(p, q, wrel, wroot, b1row, cnt)


def _rot64(v):
    return jnp.concatenate([v[:, 64:128], v[:, 0:64]], axis=1)


def _gates(ginp, h, c, wh):
    g = ginp + ((wh[0] * h[0:1] + wh[1] * h[1:2]) + wh[2] * h[2:3])
    sg = jax.nn.sigmoid(g)
    c2 = sg[3:6] * c + sg[0:3] * jnp.tanh(g[6:9])
    h2 = sg[9:12] * jnp.tanh(c2)
    return h2, c2


def _lstm_body(x_ref, wi0_ref, wh0_ref, b0_ref, wi1_ref, wh1_ref, b1_ref,
               mw_ref, mb_ref, o_ref, a_sc, b2_sc, b_sc):
    wi0 = [wi0_ref[i] for i in range(2)]
    wh0 = [wh0_ref[k] for k in range(3)]
    b0 = b0_ref[...]
    lanes = lax.broadcasted_iota(jnp.int32, (3, 128), 1)
    mask3 = lanes < 64

    def step0(t, carry):
        h, c = carry
        xc = x_ref[t]
        ginp = (b0 + wi0[0] * xc[0:1, :]) + wi0[1] * xc[1:2, :]
        h2, c2 = _gates(ginp, h, c, wh0)
        a_sc[t] = h2
        b2_sc[SL - 1 - t] = h2
        return (h2, c2)

    z = jnp.zeros((3, 128), jnp.float32)
    lax.fori_loop(0, SL, step0, (z, z))

    def swap(i, carry):
        blk = b2_sc[pl.ds(i * 64, 64)]
        b_sc[pl.ds(i * 64, 64)] = jnp.concatenate(
            [blk[:, :, 64:128], blk[:, :, 0:64]], axis=2)
        return carry
    lax.fori_loop(0, 40, swap, 0)

    wi1 = [wi1_ref[i] for i in range(6)]
    wh1 = [wh1_ref[k] for k in range(3)]
    b1 = b1_ref[...]

    def step1(t, carry):
        h, c, acc = carry
        av = a_sc[t]
        bv = b_sc[t]
        xca = jnp.where(mask3, av, bv)
        xcb = jnp.where(mask3, bv, av)
        ginp = ((b1 + wi1[0] * xca[0:1]) + (wi1[1] * xca[1:2] + wi1[2] * xca[2:3])
                + ((wi1[3] * xcb[0:1] + wi1[4] * xcb[1:2]) + wi1[5] * xcb[2:3]))
        h2, c2 = _gates(ginp, h, c, wh1)
        return (h2, c2, acc + h2 * mw_ref[t])

    _, _, acc = lax.fori_loop(0, SL, step1, (z, z, z))
    tot = acc[0:1] + acc[1:2] + acc[2:3]
    o_ref[...] = tot[:, 0:64] + tot[:, 64:128] + mb_ref[...]


def _lstm_all(xcat0, wi0, wh0, b0, wi1, wh1, b1, mw, mb):
    return pl.pallas_call(
        _lstm_body,
        out_shape=jax.ShapeDtypeStruct((1, 64), jnp.float32),
        scratch_shapes=[
            pltpu.VMEM((SL, 3, 128), jnp.float32),
            pltpu.VMEM((2560, 3, 128), jnp.float32),
            pltpu.VMEM((2560, 3, 128), jnp.float32),
        ],
    )(xcat0, wi0, wh0, b0, wi1, wh1, b1, mw, mb)


def _dir_pair(pf, pr, key, rows):
    f = jnp.broadcast_to(pf[key].T[:, :, None], (pf[key].shape[1], rows, 64))
    r = jnp.broadcast_to(pr[key].T[:, :, None], (pr[key].shape[1], rows, 64))
    return jnp.concatenate([f, r], axis=2)


def _bias_pair(pf, pr):
    bf = jnp.broadcast_to((pf["bih"] + pf["bhh"])[:, None], (12, 64))
    br = jnp.broadcast_to((pr["bih"] + pr["bhh"])[:, None], (12, 64))
    return jnp.concatenate([bf, br], axis=1)


def kernel(x, edge_index, edge_attr, batch, params):
    src = edge_index[0].astype(jnp.int32)
    dst = edge_index[1].astype(jnp.int32)
    batch = batch.astype(jnp.int32)

    a2 = _build_edge_matrix()(src, dst, edge_attr, batch)
    return _tail(a2, x, batch, params)


def _tail(a2, x, batch, params):
    a2 = a2.reshape(2, NB, NN)

    batch2d = jnp.broadcast_to(batch[None, :], (8, NN))
    p, q, cnt = _pool_mm(a2, batch2d, x)
    pooled_t = _head_mm(p, q, params["W1_rel"], params["W1_root"],
                        params["b1_rel"][None, :], cnt)

    seq = pooled_t.reshape(SL, 2, 64)
    xcat0 = jnp.concatenate([seq, seq[::-1]], axis=2)

    lp = params["lstm"]
    wi0 = _dir_pair(lp["l0"]["fwd"], lp["l0"]["rev"], "Wih", 12)
    wh0 = _dir_pair(lp["l0"]["fwd"], lp["l0"]["rev"], "Whh", 12)
    b0 = _bias_pair(lp["l0"]["fwd"], lp["l0"]["rev"])
    wi1 = _dir_pair(lp["l1"]["fwd"], lp["l1"]["rev"], "Wih", 12)
    wh1 = _dir_pair(lp["l1"]["fwd"], lp["l1"]["rev"], "Whh", 12)
    b1 = _bias_pair(lp["l1"]["fwd"], lp["l1"]["rev"])

    m6 = params["mlp_W"].reshape(SL, 6)
    mwf = jnp.broadcast_to(m6[:, 0:3][:, :, None], (SL, 3, 64))
    mwr = jnp.broadcast_to(m6[::-1, 3:6][:, :, None], (SL, 3, 64))
    mw = jnp.concatenate([mwf, mwr], axis=2)
    mb = jnp.broadcast_to(params["mlp_b"][None, :], (1, 64))

    res = _lstm_all(xcat0, wi0, wh0, b0, wi1, wh1, b1, mw, mb)
    return res.reshape(NB, 1)
